# Initial kernel scaffold; baseline (speedup 1.0000x reference)
import jax
import jax.numpy as jnp
from jax.experimental import pallas as pl


def kernel(values, mch_sorted_raw_ids, mch_remapped_ids_mapping):
    raise NotImplementedError("write your pallas kernel here")


# speedup vs baseline: 24.8347x; 24.8347x over previous
import functools

import jax
import jax.numpy as jnp
from jax import lax
from jax.experimental import pallas as pl
from jax.experimental.pallas import tpu as pltpu
from jax.experimental.pallas import tpu_sc as plsc

ZCH_SIZE = 1000000
HASH_SIZE = 100000
I32MAX = 2**31 - 1

N_VALUES = 819200
NW = 32
PER_W = N_VALUES // NW
CHUNK = 2560
NCHUNK = PER_W // CHUNK
NGROUP = CHUNK // 16
SEG = 128
NSEG = CHUNK // SEG
NBUCKET = 62501
LVL1 = 65536

_mesh = plsc.VectorSubcoreMesh(core_axis_name="c", subcore_axis_name="s")


@functools.partial(
    pl.kernel,
    mesh=_mesh,
    compiler_params=pltpu.CompilerParams(
        needs_layout_passes=False, use_tc_tiling_on_sc=False),
    out_type=jax.ShapeDtypeStruct((N_VALUES,), jnp.int32),
    scratch_types=[
        pltpu.VMEM((LVL1,), jnp.int32),
        pltpu.VMEM((CHUNK,), jnp.int32),
        pltpu.VMEM((CHUNK,), jnp.int32),
        pltpu.VMEM((CHUNK, 16), jnp.int32),
        pltpu.VMEM((CHUNK,), jnp.int32),
        pltpu.VMEM((CHUNK,), jnp.int32),
        pltpu.VMEM((CHUNK,), jnp.int32),
        pltpu.VMEM((CHUNK,), jnp.int32),
        pltpu.SemaphoreType.DMA,
        pltpu.SemaphoreType.DMA,
    ],
)
def _remap_sc(vals_hbm, lvl1_hbm, buckets_hbm, map_hbm, out_hbm,
              lvl1_v, vals_v, bidx_v, rows_v, fidx_v, match_v, mapped_v,
              out_v, sem_rows, sem_map):
    i32 = jnp.int32
    cid = lax.axis_index("c")
    sid = lax.axis_index("s")
    wid = sid * i32(2) + cid
    pltpu.sync_copy(lvl1_hbm, lvl1_v)
    iota = lax.iota(jnp.int32, 16)

    def chunk_body(ch, carry):
        base = pl.multiple_of(wid * i32(PER_W) + ch * i32(CHUNK), SEG)
        pltpu.sync_copy(vals_hbm.at[pl.ds(base, CHUNK)], vals_v)

        def bs_body(g, carry):
            v = vals_v[pl.ds(g * i32(16), 16)]
            pos = jnp.zeros((16,), jnp.int32)
            for p in (1 << k for k in range(15, -1, -1)):
                probe = plsc.load_gather(lvl1_v, [pos + (p - 1)])
                pos = jnp.where(probe < v, pos + p, pos)
            bidx_v[pl.ds(g * i32(16), 16)] = jnp.maximum(pos - 1, 0)
            return carry

        lax.fori_loop(jnp.int32(0), jnp.int32(NGROUP), bs_body, None)

        copies = [
            pltpu.async_copy(
                buckets_hbm.at[bidx_v.at[pl.ds(s * SEG, SEG)]],
                rows_v.at[pl.ds(s * SEG, SEG)],
                sem_rows,
            )
            for s in range(NSEG)
        ]
        for c in copies:
            c.wait()

        def fine_body(g, carry):
            v = vals_v[pl.ds(g * i32(16), 16)]
            b = bidx_v[pl.ds(g * i32(16), 16)]
            row_i = g * i32(16) + iota
            cnt = jnp.zeros((16,), jnp.int32)
            eqa = jnp.zeros((16,), jnp.int32)
            for t in range(16):
                col = plsc.load_gather(
                    rows_v, [row_i, jnp.full((16,), t, jnp.int32)])
                cnt = cnt + (col < v).astype(jnp.int32)
                eqa = eqa | (col == v).astype(jnp.int32)
            probe2 = plsc.load_gather(lvl1_v, [b + 1])
            m = jnp.where(cnt == 16, (probe2 == v).astype(jnp.int32), eqa)
            fidx_v[pl.ds(g * i32(16), 16)] = b * i32(16) + cnt
            match_v[pl.ds(g * i32(16), 16)] = m
            return carry

        lax.fori_loop(jnp.int32(0), jnp.int32(NGROUP), fine_body, None)

        copies = [
            pltpu.async_copy(
                map_hbm.at[fidx_v.at[pl.ds(s * SEG, SEG)]],
                mapped_v.at[pl.ds(s * SEG, SEG)],
                sem_map,
            )
            for s in range(NSEG)
        ]
        for c in copies:
            c.wait()

        def comb_body(g, carry):
            v = vals_v[pl.ds(g * i32(16), 16)]
            m = match_v[pl.ds(g * i32(16), 16)]
            mp = mapped_v[pl.ds(g * i32(16), 16)]
            h = v % HASH_SIZE + ZCH_SIZE
            out_v[pl.ds(g * i32(16), 16)] = jnp.where(m != 0, mp, h)
            return carry

        lax.fori_loop(jnp.int32(0), jnp.int32(NGROUP), comb_body, None)
        pltpu.sync_copy(out_v, out_hbm.at[pl.ds(base, CHUNK)])
        return carry

    lax.fori_loop(jnp.int32(0), jnp.int32(NCHUNK), chunk_body, None)


@jax.jit
def kernel(values, mch_sorted_raw_ids, mch_remapped_ids_mapping):
    v32 = values.astype(jnp.int32)
    ids32 = jnp.clip(mch_sorted_raw_ids, 0, I32MAX).astype(jnp.int32)
    ids_pad = jnp.concatenate(
        [ids32, jnp.full((NBUCKET * 16 - ZCH_SIZE - 1,), I32MAX, jnp.int32)])
    buckets = ids_pad.reshape(NBUCKET, 16)
    lvl1 = jnp.concatenate(
        [buckets[:, 0], jnp.full((LVL1 - NBUCKET,), I32MAX, jnp.int32)])
    map_pad = jnp.concatenate(
        [mch_remapped_ids_mapping.astype(jnp.int32),
         jnp.zeros((16,), jnp.int32)])
    out32 = _remap_sc(v32, lvl1, buckets, map_pad)
    return out32.astype(jnp.int64)
